# baseline (speedup 1.0000x reference)
import functools

import jax
import jax.numpy as jnp
from jax import lax
from jax.experimental import pallas as pl
from jax.experimental.pallas import tpu as pltpu
from jax.experimental.pallas import tpu_sc as plsc

B, F, L, V, D = 4096, 26, 20, 100000, 32
NC, NS = 2, 16
NW = NC * NS
UNITS = F * D
UNITS_W = UNITS // NW
BB = 512
NBLK = B // BB
NGRP = BB // 16
VMAIN = (V // 128) * 128
VTAIL = V - VMAIN


def _sc_pooled_lookup_t(tbl_hbm, tail_hbm, idx_hbm, out_hbm,
                        slice_v, idxv0, idxv1, outv,
                        sem_s, sem_i0, sem_i1, sem_o):
    wid = lax.axis_index("s") * NC + lax.axis_index("c")
    idxv = (idxv0, idxv1)
    sem_i = (sem_i0, sem_i1)

    def idx_copy(f, blk, ib):
        return pltpu.make_async_copy(
            idx_hbm.at[f, :, pl.ds(blk * BB, BB)], idxv[ib], sem_i[ib])

    def out_copy(u):
        return pltpu.make_async_copy(outv, out_hbm.at[u], sem_o)

    _SPLITS = (0, 196 * 128, 391 * 128, 586 * 128, VMAIN)

    def slice_copies(f, d, u):
        cps = [
            pltpu.make_async_copy(
                tbl_hbm.at[f, d, pl.ds(lo, hi - lo)],
                slice_v.at[pl.ds(lo, hi - lo)], sem_s)
            for lo, hi in zip(_SPLITS[:-1], _SPLITS[1:])
        ]
        cps.append(pltpu.make_async_copy(tail_hbm.at[u],
                                         slice_v.at[pl.ds(VMAIN, 128)],
                                         sem_s))
        return cps

    def process_blocks(f):
        def blk_pair(bb_i, carry):
            for ib in range(2):
                blk = 2 * bb_i + ib
                if ib == 0:
                    idx_copy(f, blk + 1, 1).start()
                else:
                    @pl.when(bb_i < NBLK // 2 - 1)
                    def _start_next():
                        idx_copy(f, blk + 1, 0).start()
                idx_copy(f, blk, ib).wait()
                ivb = idxv[ib]
                ovb = outv

                def grp_body(grp, _g):
                    base = grp * 16
                    acc0 = plsc.load_gather(slice_v, [ivb[0, pl.ds(base, 16)]])
                    acc1 = plsc.load_gather(slice_v, [ivb[1, pl.ds(base, 16)]])
                    for l in range(2, L, 2):
                        acc0 = acc0 + plsc.load_gather(
                            slice_v, [ivb[l, pl.ds(base, 16)]])
                        acc1 = acc1 + plsc.load_gather(
                            slice_v, [ivb[l + 1, pl.ds(base, 16)]])
                    ovb[pl.ds(blk * BB + base, 16)] = acc0 + acc1
                    return _g

                lax.fori_loop(0, NGRP, grp_body, 0, unroll=8)
            return carry

        lax.fori_loop(0, NBLK // 2, blk_pair, 0, unroll=False)

    def unit_body(k, carry):
        u = wid * UNITS_W + k
        f = u // D
        d = u - f * D

        cps = slice_copies(f, d, u)
        for cp in cps:
            cp.start()
        idx_copy(f, 0, 0).start()

        @pl.when(k >= 1)
        def _drain_prev():
            out_copy(u - 1).wait()

        for cp in cps:
            cp.wait()
        process_blocks(f)
        out_copy(u).start()
        return carry

    lax.fori_loop(0, UNITS_W, unit_body, 0, unroll=False)
    out_copy(wid * UNITS_W + UNITS_W - 1).wait()


@jax.jit
def kernel(indices, tables):
    tbl_t = tables.transpose(0, 2, 1)
    idx_t = indices.astype(jnp.int32).transpose(1, 2, 0)
    tail = jnp.pad(tbl_t[:, :, VMAIN:], ((0, 0), (0, 0), (0, 128 - VTAIL)))
    tail = tail.reshape(UNITS, 128)

    mesh = plsc.VectorSubcoreMesh(core_axis_name="c", subcore_axis_name="s")
    run = functools.partial(
        pl.kernel,
        out_type=jax.ShapeDtypeStruct((UNITS, B), jnp.float32),
        mesh=mesh,
        compiler_params=pltpu.CompilerParams(needs_layout_passes=False),
        scratch_types=[
            pltpu.VMEM((VMAIN + 128,), jnp.float32),
            pltpu.VMEM((L, BB), jnp.int32),
            pltpu.VMEM((L, BB), jnp.int32),
            pltpu.VMEM((B,), jnp.float32),
            pltpu.SemaphoreType.DMA,
            pltpu.SemaphoreType.DMA,
            pltpu.SemaphoreType.DMA,
            pltpu.SemaphoreType.DMA,
        ],
    )(_sc_pooled_lookup_t)
    out_t = run(tbl_t, tail, idx_t)
    return out_t.reshape(F, D, B).transpose(2, 0, 1)


# speedup vs baseline: 1.0054x; 1.0054x over previous
import functools

import jax
import jax.numpy as jnp
from jax import lax
from jax.experimental import pallas as pl
from jax.experimental.pallas import tpu as pltpu
from jax.experimental.pallas import tpu_sc as plsc

B, F, L, V, D = 4096, 26, 20, 100000, 32
NC, NS = 2, 16
NW = NC * NS
UNITS = F * D
UNITS_W = UNITS // NW
BB = 512
NBLK = B // BB
NGRP = BB // 16
VMAIN = (V // 128) * 128
VTAIL = V - VMAIN


def _sc_pooled_lookup_t(tbl_hbm, tail_hbm, idx_hbm, out_hbm,
                        slice_v, idxv0, idxv1, outv,
                        sem_s, sem_i0, sem_i1, sem_o):
    wid = lax.axis_index("s") * NC + lax.axis_index("c")
    idxv = (idxv0, idxv1)
    sem_i = (sem_i0, sem_i1)

    def idx_copy(f, blk, ib):
        return pltpu.make_async_copy(
            idx_hbm.at[f, :, pl.ds(blk * BB, BB)], idxv[ib], sem_i[ib])

    def out_copy(u):
        return pltpu.make_async_copy(outv, out_hbm.at[u], sem_o)

    _SPLITS = (0, 196 * 128, 391 * 128, 586 * 128, VMAIN)

    def slice_copies(f, d, u):
        cps = [
            pltpu.make_async_copy(
                tbl_hbm.at[f, d, pl.ds(lo, hi - lo)],
                slice_v.at[pl.ds(lo, hi - lo)], sem_s)
            for lo, hi in zip(_SPLITS[:-1], _SPLITS[1:])
        ]
        cps.append(pltpu.make_async_copy(tail_hbm.at[u],
                                         slice_v.at[pl.ds(VMAIN, 128)],
                                         sem_s))
        return cps

    def process_blocks(f):
        def blk_pair(bb_i, carry):
            for ib in range(2):
                blk = 2 * bb_i + ib
                if ib == 0:
                    idx_copy(f, blk + 1, 1).start()
                else:
                    @pl.when(bb_i < NBLK // 2 - 1)
                    def _start_next():
                        idx_copy(f, blk + 1, 0).start()
                idx_copy(f, blk, ib).wait()
                ivb = idxv[ib]
                ovb = outv

                def grp_body(grp, _g):
                    base = grp * 16
                    acc0 = plsc.load_gather(slice_v, [ivb[0, pl.ds(base, 16)]])
                    acc1 = plsc.load_gather(slice_v, [ivb[1, pl.ds(base, 16)]])
                    for l in range(2, L, 2):
                        acc0 = acc0 + plsc.load_gather(
                            slice_v, [ivb[l, pl.ds(base, 16)]])
                        acc1 = acc1 + plsc.load_gather(
                            slice_v, [ivb[l + 1, pl.ds(base, 16)]])
                    ovb[pl.ds(blk * BB + base, 16)] = acc0 + acc1
                    return _g

                lax.fori_loop(0, NGRP, grp_body, 0, unroll=4)
            return carry

        lax.fori_loop(0, NBLK // 2, blk_pair, 0, unroll=False)

    def unit_body(k, carry):
        u = wid * UNITS_W + k
        f = u // D
        d = u - f * D

        cps = slice_copies(f, d, u)
        for cp in cps:
            cp.start()
        idx_copy(f, 0, 0).start()

        @pl.when(k >= 1)
        def _drain_prev():
            out_copy(u - 1).wait()

        for cp in cps:
            cp.wait()
        process_blocks(f)
        out_copy(u).start()
        return carry

    lax.fori_loop(0, UNITS_W, unit_body, 0, unroll=False)
    out_copy(wid * UNITS_W + UNITS_W - 1).wait()


@jax.jit
def kernel(indices, tables):
    tbl_t = tables.transpose(0, 2, 1)
    idx_t = indices.astype(jnp.int32).transpose(1, 2, 0)
    tail = jnp.pad(tbl_t[:, :, VMAIN:], ((0, 0), (0, 0), (0, 128 - VTAIL)))
    tail = tail.reshape(UNITS, 128)

    mesh = plsc.VectorSubcoreMesh(core_axis_name="c", subcore_axis_name="s")
    run = functools.partial(
        pl.kernel,
        out_type=jax.ShapeDtypeStruct((UNITS, B), jnp.float32),
        mesh=mesh,
        compiler_params=pltpu.CompilerParams(needs_layout_passes=False),
        scratch_types=[
            pltpu.VMEM((VMAIN + 128,), jnp.float32),
            pltpu.VMEM((L, BB), jnp.int32),
            pltpu.VMEM((L, BB), jnp.int32),
            pltpu.VMEM((B,), jnp.float32),
            pltpu.SemaphoreType.DMA,
            pltpu.SemaphoreType.DMA,
            pltpu.SemaphoreType.DMA,
            pltpu.SemaphoreType.DMA,
        ],
    )(_sc_pooled_lookup_t)
    out_t = run(tbl_t, tail, idx_t)
    return out_t.reshape(F, D, B).transpose(2, 0, 1)


# speedup vs baseline: 1.0593x; 1.0536x over previous
import functools

import jax
import jax.numpy as jnp
from jax import lax
from jax.experimental import pallas as pl
from jax.experimental.pallas import tpu as pltpu
from jax.experimental.pallas import tpu_sc as plsc

B, F, L, V, D = 4096, 26, 20, 100000, 32
NC, NS = 2, 16
NW = NC * NS
UNITS = F * D
UNITS_W = UNITS // NW
BB = 512
NBLK = B // BB
NGRP = BB // 16
VMAIN = (V // 128) * 128
VTAIL = V - VMAIN


def _sc_pooled_lookup_t(tbl_hbm, tail_hbm, idx_hbm, out_hbm,
                        slice_v, idxv0, idxv1, outv,
                        sem_s, sem_i0, sem_i1, sem_o):
    wid = lax.axis_index("s") * NC + lax.axis_index("c")
    idxv = (idxv0, idxv1)
    sem_i = (sem_i0, sem_i1)

    def idx_copy(f, blk, ib):
        return pltpu.make_async_copy(
            idx_hbm.at[f, :, pl.ds(blk * BB, BB)], idxv[ib], sem_i[ib])

    def out_copy(u):
        return pltpu.make_async_copy(outv, out_hbm.at[u], sem_o)

    _SPLITS = (0, 196 * 128, 391 * 128, 586 * 128, VMAIN)

    def slice_copies(f, d, u):
        cps = [
            pltpu.make_async_copy(
                tbl_hbm.at[f, d, pl.ds(lo, hi - lo)],
                slice_v.at[pl.ds(lo, hi - lo)], sem_s)
            for lo, hi in zip(_SPLITS[:-1], _SPLITS[1:])
        ]
        cps.append(pltpu.make_async_copy(tail_hbm.at[u],
                                         slice_v.at[pl.ds(VMAIN, 128)],
                                         sem_s))
        return cps

    def process_blocks(f, next_f):
        def blk_pair(bb_i, carry):
            for ib in range(2):
                blk = 2 * bb_i + ib
                if ib == 0:
                    idx_copy(f, blk + 1, 1).start()
                else:
                    @pl.when(bb_i < NBLK // 2 - 1)
                    def _start_next():
                        idx_copy(f, blk + 1, 0).start()

                    @pl.when(bb_i == NBLK // 2 - 1)
                    def _prefetch_next_unit():
                        idx_copy(next_f, 0, 0).start()
                idx_copy(f, blk, ib).wait()
                ivb = idxv[ib]
                ovb = outv

                def grp_body(grp, _g):
                    base = grp * 16
                    acc0 = plsc.load_gather(slice_v, [ivb[0, pl.ds(base, 16)]])
                    acc1 = plsc.load_gather(slice_v, [ivb[1, pl.ds(base, 16)]])
                    for l in range(2, L, 2):
                        acc0 = acc0 + plsc.load_gather(
                            slice_v, [ivb[l, pl.ds(base, 16)]])
                        acc1 = acc1 + plsc.load_gather(
                            slice_v, [ivb[l + 1, pl.ds(base, 16)]])
                    ovb[pl.ds(blk * BB + base, 16)] = acc0 + acc1
                    return _g

                lax.fori_loop(0, NGRP, grp_body, 0, unroll=4)
            return carry

        lax.fori_loop(0, NBLK // 2, blk_pair, 0, unroll=False)

    def next_f_of(u):
        return jnp.minimum((u + 1) // D, F - 1)

    def unit_body(k, carry):
        u = wid * UNITS_W + k
        f = u // D
        d = u - f * D

        cps = slice_copies(f, d, u)
        for cp in cps:
            cp.start()

        @pl.when(k >= 1)
        def _drain_prev():
            out_copy(u - 1).wait()

        for cp in cps:
            cp.wait()
        process_blocks(f, next_f_of(u))
        out_copy(u).start()
        return carry

    first_u = wid * UNITS_W
    idx_copy(first_u // D, 0, 0).start()
    lax.fori_loop(0, UNITS_W, unit_body, 0, unroll=False)
    last_u = wid * UNITS_W + UNITS_W - 1
    idx_copy(next_f_of(last_u), 0, 0).wait()
    out_copy(last_u).wait()


@jax.jit
def kernel(indices, tables):
    tbl_t = tables.transpose(0, 2, 1)
    idx_t = indices.astype(jnp.int32).transpose(1, 2, 0)
    tail = jnp.pad(tbl_t[:, :, VMAIN:], ((0, 0), (0, 0), (0, 128 - VTAIL)))
    tail = tail.reshape(UNITS, 128)

    mesh = plsc.VectorSubcoreMesh(core_axis_name="c", subcore_axis_name="s")
    run = functools.partial(
        pl.kernel,
        out_type=jax.ShapeDtypeStruct((UNITS, B), jnp.float32),
        mesh=mesh,
        compiler_params=pltpu.CompilerParams(needs_layout_passes=False),
        scratch_types=[
            pltpu.VMEM((VMAIN + 128,), jnp.float32),
            pltpu.VMEM((L, BB), jnp.int32),
            pltpu.VMEM((L, BB), jnp.int32),
            pltpu.VMEM((B,), jnp.float32),
            pltpu.SemaphoreType.DMA,
            pltpu.SemaphoreType.DMA,
            pltpu.SemaphoreType.DMA,
            pltpu.SemaphoreType.DMA,
        ],
    )(_sc_pooled_lookup_t)
    out_t = run(tbl_t, tail, idx_t)
    return out_t.reshape(F, D, B).transpose(2, 0, 1)
